# Initial kernel scaffold; baseline (speedup 1.0000x reference)
import jax
import jax.numpy as jnp
from jax.experimental import pallas as pl


def kernel(image_shape, anchors, rpn_score, rpn_bboxes_txtytwth, gt_bboxes):
    raise NotImplementedError("write your pallas kernel here")


# speedup vs baseline: 7.1571x; 7.1571x over previous
import functools

import jax
import jax.numpy as jnp
from jax.experimental import pallas as pl
from jax.experimental.pallas import tpu as pltpu

_POS_TH = 0.7
_NEG_TH = 0.3
_TOTAL = 256
_MAX_POS = 128
_R = 160
_C = 128
_NPAD = _R * _C


def _rpn_kernel(n_anchors, n_gt, img_ref, gt_ref, a_ref, s_ref, p_ref,
                cls_ref, reg_ref, iou_ref):
    f32 = jnp.float32
    H = img_ref[0]
    W = img_ref[1]
    ax1 = a_ref[0]
    ay1 = a_ref[1]
    ax2 = a_ref[2]
    ay2 = a_ref[3]
    area_a = (ax2 - ax1) * (ay2 - ay1)

    max_iou = jnp.zeros((_R, _C), f32)
    arg = jnp.zeros((_R, _C), jnp.int32)
    bests = []
    for j in range(n_gt):
        bx1 = gt_ref[j, 0]
        by1 = gt_ref[j, 1]
        bx2 = gt_ref[j, 2]
        by2 = gt_ref[j, 3]
        area_b = (bx2 - bx1) * (by2 - by1)
        ix1 = jnp.maximum(ax1, bx1)
        iy1 = jnp.maximum(ay1, by1)
        ix2 = jnp.minimum(ax2, bx2)
        iy2 = jnp.minimum(ay2, by2)
        iw = jnp.maximum(ix2 - ix1, 0.0)
        ih = jnp.maximum(iy2 - iy1, 0.0)
        inter = iw * ih
        union = area_a + area_b - inter
        iou = inter / jnp.maximum(union, 1e-8)
        iou_ref[j] = iou
        bests.append(jnp.max(iou))
        upd = iou > max_iou
        arg = jnp.where(upd, j, arg)
        max_iou = jnp.where(upd, iou, max_iou)

    is_best_any = jnp.zeros((_R, _C), jnp.bool_)
    mgx1 = jnp.zeros((_R, _C), f32)
    mgy1 = jnp.zeros((_R, _C), f32)
    mgx2 = jnp.zeros((_R, _C), f32)
    mgy2 = jnp.zeros((_R, _C), f32)
    for j in range(n_gt):
        iou = iou_ref[j]
        is_best_any = is_best_any | (iou >= bests[j] - 1e-12)
        m = arg == j
        mgx1 = jnp.where(m, gt_ref[j, 0], mgx1)
        mgy1 = jnp.where(m, gt_ref[j, 1], mgy1)
        mgx2 = jnp.where(m, gt_ref[j, 2], mgx2)
        mgy2 = jnp.where(m, gt_ref[j, 3], mgy2)

    row = jax.lax.broadcasted_iota(jnp.int32, (_R, _C), 0)
    col = jax.lax.broadcasted_iota(jnp.int32, (_R, _C), 1)
    valid = (row * _C + col) < n_anchors
    inside = (ax1 >= 0.0) & (ay1 >= 0.0) & (ax2 <= W) & (ay2 <= H)
    is_best = is_best_any & (max_iou > 0.0)
    pos = valid & inside & ((max_iou >= _POS_TH) | is_best)
    neg = valid & inside & (max_iou < _NEG_TH) & jnp.logical_not(pos)
    posf = pos.astype(f32)
    negf = neg.astype(f32)
    pos_count = jnp.sum(posf)
    neg_count = jnp.sum(negf)

    bits = jax.lax.bitcast_convert_type(max_iou, jnp.int32)
    key_bits = jnp.where(pos, bits, -1)

    def bs_body(_, carry):
        lo, hi = carry
        mid = lo + (hi - lo) // 2
        cnt = jnp.sum((key_bits > mid).astype(f32))
        take = cnt < float(_MAX_POS)
        return jnp.where(take, lo, mid), jnp.where(take, mid, hi)

    lo0 = jnp.int32(-1)
    hi0 = jnp.int32(0x40000000)
    _, B = jax.lax.fori_loop(0, 31, bs_body, (lo0, hi0))
    cnt_gt = jnp.sum((key_bits > B).astype(f32))
    need_eq = float(_MAX_POS) - cnt_gt
    eq = pos & (key_bits == B)

    ur = jax.lax.broadcasted_iota(jnp.int32, (_C, _C), 0)
    uc = jax.lax.broadcasted_iota(jnp.int32, (_C, _C), 1)
    U = (ur <= uc).astype(f32)
    lr = jax.lax.broadcasted_iota(jnp.int32, (_R, _R), 0)
    lc = jax.lax.broadcasted_iota(jnp.int32, (_R, _R), 1)
    L = (lc < lr).astype(f32)

    def excl_prefix(mf):
        incl = jnp.dot(mf, U, preferred_element_type=f32)
        rowtot = incl[:, _C - 1:_C]
        rows_excl = jnp.dot(L, rowtot, preferred_element_type=f32)
        return rows_excl + (incl - mf)

    eq_rank = excl_prefix(eq.astype(f32))
    take_all = pos_count <= float(_MAX_POS)
    sel_pos = pos & (take_all | (key_bits > B) | (eq & (eq_rank < need_eq)))
    pos_num = jnp.minimum(pos_count, float(_MAX_POS))

    neg_rank = excl_prefix(negf)
    neg_num = jnp.minimum(float(_TOTAL) - pos_num, neg_count)
    sel_neg = neg & (neg_rank < neg_num)
    total = pos_num + neg_num

    s0 = s_ref[0]
    s1 = s_ref[1]
    mm = jnp.maximum(s0, s1)
    lse = mm + jnp.log(jnp.exp(s0 - mm) + jnp.exp(s1 - mm))
    cls_sum = (jnp.sum(jnp.where(sel_pos, lse - s1, 0.0)) +
               jnp.sum(jnp.where(sel_neg, lse - s0, 0.0)))
    cls_loss = cls_sum / total

    aw = jnp.maximum(ax2 - ax1, 1e-3)
    ah = jnp.maximum(ay2 - ay1, 1e-3)
    acx = ax1 + 0.5 * aw
    acy = ay1 + 0.5 * ah
    gw = jnp.maximum(mgx2 - mgx1, 1e-3)
    gh = jnp.maximum(mgy2 - mgy1, 1e-3)
    gcx = mgx1 + 0.5 * gw
    gcy = mgy1 + 0.5 * gh
    tx = ((gcx - acx) / aw) / 0.1
    ty = ((gcy - acy) / ah) / 0.1
    tw = jnp.log(gw / aw) / 0.2
    th = jnp.log(gh / ah) / 0.2

    def sl1(d):
        ad = jnp.abs(d)
        return jnp.where(ad < 1.0 / 9.0, 0.5 * 9.0 * d * d, ad - 0.5 / 9.0)

    l = (sl1(p_ref[0] - tx) + sl1(p_ref[1] - ty) +
         sl1(p_ref[2] - tw) + sl1(p_ref[3] - th))
    reg_sum = jnp.sum(jnp.where(sel_pos, l, 0.0))
    reg = reg_sum / total
    reg_loss = jnp.where(pos_num == 0.0, 0.0, reg)

    cls_ref[0] = cls_loss
    reg_ref[0] = reg_loss


def kernel(image_shape, anchors, rpn_score, rpn_bboxes_txtytwth, gt_bboxes):
    n = anchors.shape[0]
    n_gt = gt_bboxes.shape[0]
    f32 = jnp.float32

    def planes(x):
        xt = jnp.transpose(x.astype(f32))
        return jnp.pad(xt, ((0, 0), (0, _NPAD - n))).reshape(-1, _R, _C)

    a_t = planes(anchors)
    s_t = planes(rpn_score)
    p_t = planes(rpn_bboxes_txtytwth)

    body = functools.partial(_rpn_kernel, n, n_gt)
    cls_out, reg_out = pl.pallas_call(
        body,
        out_shape=[jax.ShapeDtypeStruct((1,), f32),
                   jax.ShapeDtypeStruct((1,), f32)],
        in_specs=[
            pl.BlockSpec(memory_space=pltpu.SMEM),
            pl.BlockSpec(memory_space=pltpu.SMEM),
            pl.BlockSpec(memory_space=pltpu.VMEM),
            pl.BlockSpec(memory_space=pltpu.VMEM),
            pl.BlockSpec(memory_space=pltpu.VMEM),
        ],
        out_specs=[
            pl.BlockSpec(memory_space=pltpu.SMEM),
            pl.BlockSpec(memory_space=pltpu.SMEM),
        ],
        scratch_shapes=[pltpu.VMEM((n_gt, _R, _C), f32)],
    )(image_shape.astype(f32), gt_bboxes.astype(f32), a_t, s_t, p_t)
    return (cls_out[0], reg_out[0])
